# baseline (speedup 1.0000x reference)
import functools

import jax
import jax.numpy as jnp
from jax.experimental import pallas as pl
from jax.experimental.pallas import tpu as pltpu

_BLK = 200


def _pass1_body(x_ref, w1_ref, b1_ref, w2_ref, adj_ref,
                adjb_ref, s2_ref, s1_ref, *, nblk, blk):
    i = pl.program_id(0)

    @pl.when(i == 0)
    def _prologue():
        s1_ref[...] = jnp.dot(x_ref[...], w1_ref[...],
                              preferred_element_type=jnp.float32)

    @pl.when(i >= 1)
    def _pass1():
        a = adj_ref[...]
        h = jnp.dot(a, s1_ref[...], preferred_element_type=jnp.float32)
        h = jnp.maximum(h + b1_ref[...], 0.0)
        s2_ref[...] = jnp.dot(h, w2_ref[...],
                              preferred_element_type=jnp.float32)
        adjb_ref[...] = a.astype(jnp.bfloat16)


def _pass2_body(adjb_ref, s2_ref, b2_ref, o_ref):
    o = jnp.dot(adjb_ref[...], s2_ref[...],
                preferred_element_type=jnp.float32)
    o = o + b2_ref[...]
    m = jnp.max(o, axis=1, keepdims=True)
    e = o - m
    o_ref[...] = e - jnp.log(jnp.sum(jnp.exp(e), axis=1, keepdims=True))


def kernel(x, adj, W1, b1, W2, b2):
    n, nfeat = x.shape
    nhid = W1.shape[1]
    ncls = W2.shape[1]
    blk = _BLK
    nblk = n // blk

    body1 = functools.partial(_pass1_body, nblk=nblk, blk=blk)

    def ridx(i):
        return (jnp.maximum(i - 1, 0), 0)

    adjb, s2 = pl.pallas_call(
        body1,
        grid=(1 + nblk,),
        in_specs=[
            pl.BlockSpec((n, nfeat), lambda i: (0, 0)),
            pl.BlockSpec((nfeat, nhid), lambda i: (0, 0)),
            pl.BlockSpec((1, nhid), lambda i: (0, 0)),
            pl.BlockSpec((nhid, ncls), lambda i: (0, 0)),
            pl.BlockSpec((blk, n), ridx),
        ],
        out_specs=[
            pl.BlockSpec((blk, n), ridx),
            pl.BlockSpec((blk, ncls), ridx),
        ],
        out_shape=[
            jax.ShapeDtypeStruct((n, n), jnp.bfloat16),
            jax.ShapeDtypeStruct((n, ncls), jnp.float32),
        ],
        scratch_shapes=[
            pltpu.VMEM((n, nhid), jnp.float32),
        ],
        compiler_params=pltpu.CompilerParams(
            dimension_semantics=("arbitrary",),
            vmem_limit_bytes=67108864,
        ),
    )(x, W1, b1.reshape(1, nhid), W2, adj)

    return pl.pallas_call(
        _pass2_body,
        grid=(nblk,),
        in_specs=[
            pl.BlockSpec((blk, n), lambda i: (i, 0)),
            pl.BlockSpec((n, ncls), lambda i: (0, 0)),
            pl.BlockSpec((1, ncls), lambda i: (0, 0)),
        ],
        out_specs=pl.BlockSpec((blk, ncls), lambda i: (i, 0)),
        out_shape=jax.ShapeDtypeStruct((n, ncls), jnp.float32),
        compiler_params=pltpu.CompilerParams(
            dimension_semantics=("arbitrary",),
            vmem_limit_bytes=67108864,
        ),
    )(adjb, s2, b2.reshape(1, ncls))
